# Initial kernel scaffold; baseline (speedup 1.0000x reference)
import jax
import jax.numpy as jnp
from jax.experimental import pallas as pl


def kernel(x, pos, batch, params):
    raise NotImplementedError("write your pallas kernel here")


# speedup vs baseline: 1.4404x; 1.4404x over previous
import functools

import jax
import jax.numpy as jnp
from jax import lax
from jax.experimental import pallas as pl
from jax.experimental.pallas import tpu as pltpu
from jax.experimental.pallas import tpu_sc as plsc

N = 10000
NP = 10240
G = 16
KN = 3
EP = KN * NP
E_REAL = KN * N

RB = 256
CT = 256
NRB = NP // RB

RBE = 512
NPB = NP // RBE

RBA = 256
NPA = NP // RBA

MASKV = 1e30
BIGV = 3e38
IBIG = 2**30

SC_NC = 2
SC_NS = 16
SC_NW = SC_NC * SC_NS
SC_CH = 120
SC_NCH = EP // SC_NW // SC_CH


def _knn_body(lo_ref, hi_ref, feat_ref, batch_ref, nbr_ref):
    rb = pl.program_id(0)
    rows = feat_ref[pl.ds(rb * RB, RB), :]
    sq_i = jnp.sum(rows * rows, axis=1)
    batch_rows = batch_ref[pl.ds(rb * RB, RB)]
    rowid = rb * RB + lax.broadcasted_iota(jnp.int32, (CT, RB), 1)

    def tile(j, carry):
        v0, i0, v1, i1, v2, i2 = carry
        cols = feat_ref[pl.ds(j * CT, CT), :]
        dots = lax.dot_general(cols, rows, (((1,), (1,)), ((), ())),
                               preferred_element_type=jnp.float32)
        sq_j = jnp.sum(cols * cols, axis=1)[:, None]
        bc = batch_ref[pl.ds(j * CT, CT)][:, None]
        d2 = sq_i[None, :] + sq_j - 2.0 * dots
        colid = j * CT + lax.broadcasted_iota(jnp.int32, (CT, RB), 0)
        invalid = (bc != batch_rows[None, :]) | (colid == rowid)
        d2 = jnp.where(invalid, MASKV, d2)
        for _ in range(3):
            m = jnp.min(d2, axis=0)
            cand = jnp.where(d2 <= m[None, :], colid, IBIG)
            a = jnp.min(cand, axis=0)
            d2 = jnp.where(colid == a[None, :], BIGV, d2)
            lt2 = (m < v2) | ((m == v2) & (a < i2))
            lt1 = (m < v1) | ((m == v1) & (a < i1))
            lt0 = (m < v0) | ((m == v0) & (a < i0))
            v2 = jnp.where(lt2, jnp.where(lt1, v1, m), v2)
            i2 = jnp.where(lt2, jnp.where(lt1, i1, a), i2)
            v1 = jnp.where(lt1, jnp.where(lt0, v0, m), v1)
            i1 = jnp.where(lt1, jnp.where(lt0, i0, a), i1)
            v0 = jnp.where(lt0, m, v0)
            i0 = jnp.where(lt0, a, i0)
        return (v0, i0, v1, i1, v2, i2)

    zf = jnp.full((RB,), BIGV, jnp.float32)
    zi = jnp.zeros((RB,), jnp.int32)
    v0, i0, v1, i1, v2, i2 = lax.fori_loop(
        lo_ref[rb], hi_ref[rb], tile, (zf, zi, zf, zi, zf, zi))
    r = lax.broadcasted_iota(jnp.int32, (8, RB), 0)
    nbr_ref[...] = (jnp.where(r == 0, i0[None, :], 0)
                    + jnp.where(r == 1, i1[None, :], 0)
                    + jnp.where(r == 2, i2[None, :], 0))


def _knn(feat, batch_p, lo, hi):
    d = feat.shape[1]
    return pl.pallas_call(
        _knn_body,
        grid=(NRB,),
        in_specs=[
            pl.BlockSpec(memory_space=pltpu.SMEM),
            pl.BlockSpec(memory_space=pltpu.SMEM),
            pl.BlockSpec((NP, d), lambda rb: (0, 0)),
            pl.BlockSpec((NP,), lambda rb: (0,)),
        ],
        out_specs=pl.BlockSpec((8, RB), lambda rb: (0, rb)),
        out_shape=jax.ShapeDtypeStruct((8, NP), jnp.int32),
    )(lo, hi, feat, batch_p)


def _sc_gather(table, idx2d):
    c = table.shape[1]
    mesh = plsc.VectorSubcoreMesh(core_axis_name="c", subcore_axis_name="s",
                                  num_cores=SC_NC, num_subcores=SC_NS)

    @functools.partial(
        pl.kernel,
        out_type=jax.ShapeDtypeStruct((EP, c), jnp.float32),
        mesh=mesh,
        scratch_types=[
            pltpu.VMEM((SC_NCH, SC_CH), jnp.int32),
            pltpu.VMEM((2, SC_CH, c), jnp.float32),
            pltpu.SemaphoreType.DMA,
            pltpu.SemaphoreType.DMA,
        ],
    )
    def gather_k(table_hbm, idx_hbm, out_hbm, idx_v, buf, sem0, sem1):
        wid = lax.axis_index("s") * SC_NC + lax.axis_index("c")
        pltpu.sync_copy(idx_hbm.at[pl.ds(wid * SC_NCH, SC_NCH)], idx_v)
        sems = [sem0, sem1]
        cps = [None, None]
        cps[0] = pltpu.async_copy(table_hbm.at[idx_v.at[0]], buf.at[0], sems[0])
        for j in range(SC_NCH):
            if j + 1 < SC_NCH:
                cps[(j + 1) % 2] = pltpu.async_copy(
                    table_hbm.at[idx_v.at[j + 1]], buf.at[(j + 1) % 2],
                    sems[(j + 1) % 2])
            cps[j % 2].wait()
            pltpu.sync_copy(buf.at[j % 2],
                            out_hbm.at[pl.ds((wid * SC_NCH + j) * SC_CH, SC_CH)])

    return gather_k(table, idx2d)


def _gather(table, nbr):
    idx2d = nbr[:KN].reshape(EP // SC_CH, SC_CH)
    return _sc_gather(table, idx2d).reshape(KN, NP, table.shape[1])


def _stats_update(stats_ref, h, first):
    s = jnp.sum(h, axis=0)
    s2 = jnp.sum(h * h, axis=0)
    r8 = lax.broadcasted_iota(jnp.int32, (8, h.shape[1]), 0)
    upd = (jnp.where(r8 == 0, s[None, :], 0.0)
           + jnp.where(r8 == 1, s2[None, :], 0.0))

    @pl.when(first)
    def _():
        stats_ref[...] = upd

    @pl.when(jnp.logical_not(first))
    def _():
        stats_ref[...] = stats_ref[...] + upd


def _row_mask(n, h):
    rid = n * RBE + lax.broadcasted_iota(jnp.int32, h.shape, 0)
    return jnp.where(rid < N, h, 0.0)


def _apply_bn(h, bn_ref):
    m, v, gg, be = bn_ref[0], bn_ref[1], bn_ref[2], bn_ref[3]
    return (gg[None, :] * (h - m[None, :]) * lax.rsqrt(v[None, :] + 1e-5)
            + be[None, :])


def _edge_cat(x_ref, g_ref, cw):
    xi = x_ref[...]
    diff = g_ref[0] - xi
    return jnp.concatenate([xi[:, :cw], diff[:, :cw]], axis=1)


def _mlp_first_body(x_ref, g_ref, w_ref, bv_ref, out_ref, stats_ref):
    k = pl.program_id(0)
    n = pl.program_id(1)
    e = _edge_cat(x_ref, g_ref, w_ref.shape[0] // 2)
    h = (jnp.dot(e, w_ref[...], preferred_element_type=jnp.float32)
         + bv_ref[0][None, :])
    h = _row_mask(n, jnp.maximum(h, 0.0))
    out_ref[0] = h
    _stats_update(stats_ref, h, (k == 0) & (n == 0))


def _mlp_first(x, g, w, bv):
    cin = x.shape[1]
    cout = w.shape[1]
    return pl.pallas_call(
        _mlp_first_body,
        grid=(KN, NPB),
        in_specs=[
            pl.BlockSpec((RBE, cin), lambda k, n: (n, 0)),
            pl.BlockSpec((1, RBE, cin), lambda k, n: (k, n, 0)),
            pl.BlockSpec(w.shape, lambda k, n: (0, 0)),
            pl.BlockSpec((8, cout), lambda k, n: (0, 0)),
        ],
        out_specs=[
            pl.BlockSpec((1, RBE, cout), lambda k, n: (k, n, 0)),
            pl.BlockSpec((8, cout), lambda k, n: (0, 0)),
        ],
        out_shape=[
            jax.ShapeDtypeStruct((KN, NP, cout), jnp.float32),
            jax.ShapeDtypeStruct((8, cout), jnp.float32),
        ],
    )(x, g, w, bv)


def _mlp_mid_body(in_ref, bn_ref, w_ref, bv_ref, out_ref, stats_ref):
    k = pl.program_id(0)
    n = pl.program_id(1)
    hb = _apply_bn(in_ref[0], bn_ref)
    h = (jnp.dot(hb, w_ref[...], preferred_element_type=jnp.float32)
         + bv_ref[0][None, :])
    h = _row_mask(n, jnp.maximum(h, 0.0))
    out_ref[0] = h
    _stats_update(stats_ref, h, (k == 0) & (n == 0))


def _mlp_mid(hin, bn, w, bv):
    cin = hin.shape[2]
    cout = w.shape[1]
    return pl.pallas_call(
        _mlp_mid_body,
        grid=(KN, NPB),
        in_specs=[
            pl.BlockSpec((1, RBE, cin), lambda k, n: (k, n, 0)),
            pl.BlockSpec((8, cin), lambda k, n: (0, 0)),
            pl.BlockSpec((cin, cout), lambda k, n: (0, 0)),
            pl.BlockSpec((8, cout), lambda k, n: (0, 0)),
        ],
        out_specs=[
            pl.BlockSpec((1, RBE, cout), lambda k, n: (k, n, 0)),
            pl.BlockSpec((8, cout), lambda k, n: (0, 0)),
        ],
        out_shape=[
            jax.ShapeDtypeStruct((KN, NP, cout), jnp.float32),
            jax.ShapeDtypeStruct((8, cout), jnp.float32),
        ],
    )(hin, bn, w, bv)


def _var_body(in_ref, m_ref, out_ref):
    k = pl.program_id(0)
    n = pl.program_id(1)
    d = in_ref[0] - m_ref[0][None, :]
    d = _row_mask(n, d)
    s = jnp.sum(d * d, axis=0)
    r8 = lax.broadcasted_iota(jnp.int32, (8, d.shape[1]), 0)
    upd = jnp.where(r8 == 0, s[None, :], 0.0)
    first = (k == 0) & (n == 0)

    @pl.when(first)
    def _():
        out_ref[...] = upd

    @pl.when(jnp.logical_not(first))
    def _():
        out_ref[...] = out_ref[...] + upd


def _var_pass(h, m8):
    c = h.shape[2]
    return pl.pallas_call(
        _var_body,
        grid=(KN, NPB),
        in_specs=[
            pl.BlockSpec((1, RBE, c), lambda k, n: (k, n, 0)),
            pl.BlockSpec((8, c), lambda k, n: (0, 0)),
        ],
        out_specs=pl.BlockSpec((8, c), lambda k, n: (0, 0)),
        out_shape=jax.ShapeDtypeStruct((8, c), jnp.float32),
    )(h, m8)


def _agg_c1_body(h3_ref, bn3_ref, h1_ref, bn1_ref, out_ref):
    msgs = [_apply_bn(h3_ref[k], bn3_ref) + _apply_bn(h1_ref[k], bn1_ref)
            for k in range(KN)]
    out_ref[...] = (msgs[0] + msgs[1] + msgs[2]) / 3.0


def _agg_c1(h3, bn3, h1, bn1):
    c = h3.shape[2]
    return pl.pallas_call(
        _agg_c1_body,
        grid=(NPA,),
        in_specs=[
            pl.BlockSpec((KN, RBA, c), lambda n: (0, n, 0)),
            pl.BlockSpec((8, c), lambda n: (0, 0)),
            pl.BlockSpec((KN, RBA, c), lambda n: (0, n, 0)),
            pl.BlockSpec((8, c), lambda n: (0, 0)),
        ],
        out_specs=pl.BlockSpec((RBA, c), lambda n: (n, 0)),
        out_shape=jax.ShapeDtypeStruct((NP, c), jnp.float32),
    )(h3, bn3, h1, bn1)


def _agg_res_body(h3_ref, bn3_ref, g_ref, x_ref, out_ref):
    xi = x_ref[...]
    msgs = [_apply_bn(h3_ref[k], bn3_ref)
            + jnp.concatenate([xi, g_ref[k] - xi], axis=1)
            for k in range(KN)]
    out_ref[...] = (msgs[0] + msgs[1] + msgs[2]) / 3.0


def _agg_res(h3, bn3, g, x):
    c2 = h3.shape[2]
    cn = x.shape[1]
    return pl.pallas_call(
        _agg_res_body,
        grid=(NPA,),
        in_specs=[
            pl.BlockSpec((KN, RBA, c2), lambda n: (0, n, 0)),
            pl.BlockSpec((8, c2), lambda n: (0, 0)),
            pl.BlockSpec((KN, RBA, cn), lambda n: (0, n, 0)),
            pl.BlockSpec((RBA, cn), lambda n: (n, 0)),
        ],
        out_specs=pl.BlockSpec((RBA, c2), lambda n: (n, 0)),
        out_shape=jax.ShapeDtypeStruct((NP, c2), jnp.float32),
    )(h3, bn3, g, x)


def _pool_cls_body(x_ref, batch_ref, w1_ref, misc_ref, scal_ref, out_ref,
                   pooled_ref, cnt_ref):
    n = pl.program_id(0)
    bl = batch_ref[pl.ds(n * RBA, RBA)]
    giota = lax.broadcasted_iota(jnp.int32, (G, RBA), 0)
    oh = (giota == bl[None, :]).astype(jnp.float32)
    ps = jnp.dot(oh, x_ref[...], preferred_element_type=jnp.float32,
                 precision=lax.Precision.HIGHEST)
    cnt = jnp.sum(oh, axis=1)
    r8 = lax.broadcasted_iota(jnp.int32, (8, G), 0)
    cupd = jnp.where(r8 == 0, cnt[None, :], 0.0)

    @pl.when(n == 0)
    def _():
        pooled_ref[...] = ps
        cnt_ref[...] = cupd

    @pl.when(n > 0)
    def _():
        pooled_ref[...] = pooled_ref[...] + ps
        cnt_ref[...] = cnt_ref[...] + cupd

    @pl.when(n == NPA - 1)
    def _():
        cnt_l = jnp.maximum(cnt_ref[0], 1.0)
        pooled = pooled_ref[...] / cnt_l[:, None]
        h = jnp.dot(pooled, w1_ref[...], preferred_element_type=jnp.float32)
        h = jnp.maximum(h + misc_ref[0][None, :], 0.0)
        m = jnp.mean(h, axis=0)
        v = jnp.mean(h * h, axis=0) - m * m
        h = (misc_ref[1][None, :] * (h - m[None, :])
             * lax.rsqrt(v[None, :] + 1e-5) + misc_ref[2][None, :])
        z = jnp.sum(h * misc_ref[3][None, :], axis=1) + scal_ref[0]
        z = jnp.maximum(z, 0.0)
        mz = jnp.mean(z)
        vz = jnp.mean(z * z) - mz * mz
        z = scal_ref[1] * (z - mz) * lax.rsqrt(vz + 1e-5) + scal_ref[2]
        out_ref[...] = (1.0 / (1.0 + jnp.exp(-z)))[None, :]


def _pool_cls(x3, batch_p, w1, misc, scal):
    return pl.pallas_call(
        _pool_cls_body,
        grid=(NPA,),
        in_specs=[
            pl.BlockSpec((RBA, 512), lambda n: (n, 0)),
            pl.BlockSpec((NP,), lambda n: (0,)),
            pl.BlockSpec((512, 512), lambda n: (0, 0)),
            pl.BlockSpec((8, 512), lambda n: (0, 0)),
            pl.BlockSpec(memory_space=pltpu.SMEM),
        ],
        out_specs=pl.BlockSpec((1, G), lambda n: (0, 0)),
        out_shape=jax.ShapeDtypeStruct((1, G), jnp.float32),
        scratch_shapes=[
            pltpu.VMEM((G, 512), jnp.float32),
            pltpu.VMEM((8, G), jnp.float32),
        ],
    )(x3, batch_p, w1, misc, scal)


def _bn_pack(h, stats, p):
    del stats
    hT = lax.optimization_barrier(
        h[:, :N, :].transpose(1, 0, 2).reshape(E_REAL, h.shape[2]))
    mean = jnp.mean(hT, axis=0)
    var = jnp.var(hT, axis=0)
    return _pad8(mean, var, p["g"], p["be"])


def _pad8(*rows):
    c = rows[0].shape[0]
    out = jnp.zeros((8, c), jnp.float32)
    for i, r in enumerate(rows):
        out = out.at[i].set(r)
    return out


def _edgeconv_chain(x, g, w1, b1, blocks):
    h1, st1 = _mlp_first(x, g, w1, _pad8(b1))
    bn = _bn_pack(h1, st1, blocks[0])
    bn_first = bn
    h = h1
    for i in range(1, len(blocks)):
        h, st = _mlp_mid(h, bn, blocks[i]["W"], _pad8(blocks[i]["b"]))
        bn = _bn_pack(h, st, blocks[i])
    return h1, bn_first, h, bn


def kernel(x, pos, batch, params):
    f32 = jnp.float32
    xp = jnp.zeros((NP, 128), f32).at[:N, :59].set(x.astype(f32))
    posp = jnp.zeros((NP, 8), f32).at[:N, :3].set(pos.astype(f32))
    batch_p = jnp.full((NP,), -1, jnp.int32).at[:N].set(batch.astype(jnp.int32))

    gr = jnp.arange(G, dtype=jnp.int32)
    seg_lo = jnp.searchsorted(batch, gr, side="left").astype(jnp.int32)
    seg_hi = jnp.searchsorted(batch, gr, side="right").astype(jnp.int32)
    first_b = batch[jnp.arange(NRB) * RB % N]
    last_b = batch[jnp.minimum(jnp.arange(NRB) * RB + RB - 1, N - 1)]
    lo = seg_lo[first_b] // CT
    hi = (seg_hi[last_b] + CT - 1) // CT

    p1 = params["c1_mlp"]
    nbr1 = _knn(posp, batch_p, lo, hi)
    g1 = _gather(xp, nbr1)
    h1, bn1, h3, bn3 = _edgeconv_chain(
        xp, g1, p1["W"], p1["b"], [p1] + list(params["c1_res"]))
    x1 = _agg_c1(h3, bn3, h1, bn1)

    p2 = params["c2_res"]
    nbr2 = _knn(x1, batch_p, lo, hi)
    g2 = _gather(x1, nbr2)
    _, _, h3, bn3 = _edgeconv_chain(x1, g2, p2[0]["W"], p2[0]["b"], list(p2))
    x2 = _agg_res(h3, bn3, g2, x1)

    p3 = params["c3_res"]
    nbr3 = _knn(x2, batch_p, lo, hi)
    g3 = _gather(x2, nbr3)
    _, _, h3, bn3 = _edgeconv_chain(x2, g3, p3[0]["W"], p3[0]["b"], list(p3))
    x3 = _agg_res(h3, bn3, g3, x2)

    c1, c2 = params["cls1"], params["cls2"]
    misc = _pad8(c1["b"], c1["g"], c1["be"], c2["W"][:, 0])
    scal = jnp.stack([c2["b"][0], c2["g"][0], c2["be"][0],
                      jnp.float32(0)]).astype(f32)
    out = _pool_cls(x3, batch_p, c1["W"], misc, scal)
    return out.reshape(G)
